# baseline (speedup 1.0000x reference)
import functools

import jax
import jax.numpy as jnp
from jax import lax
from jax.experimental import pallas as pl
from jax.experimental.pallas import tpu as pltpu
from jax.experimental.pallas import tpu_sc as plsc

_INFO = plsc.get_sparse_core_info()
_NC, _NS = _INFO.num_cores, _INFO.num_subcores
_NW = _NC * _NS

_CHUNK = 32
_NSLOT = 2


@functools.lru_cache(maxsize=None)
def _make_sc_broadcast(batch, seq_len, d_model):
    rows_per_w = seq_len // _NW
    assert rows_per_w * _NW == seq_len
    if rows_per_w % 48 == 32:
        sizes = [48] * (rows_per_w // 48 - 0) + [32] if rows_per_w >= 48 else [rows_per_w]
        sizes = [48] * ((rows_per_w - 32) // 48) + [32]
    else:
        c = min(_CHUNK, rows_per_w)
        sizes = [c] * (rows_per_w // c)
    offs = [sum(sizes[:k]) for k in range(len(sizes))]
    nchunk = len(sizes)
    chunk = max(sizes)
    assert sum(sizes) == rows_per_w

    mesh = plsc.VectorSubcoreMesh(core_axis_name="c", subcore_axis_name="s")

    @functools.partial(
        pl.kernel,
        mesh=mesh,
        out_type=jax.ShapeDtypeStruct((batch, seq_len, d_model), jnp.float32),
        scratch_types=[
            pltpu.VMEM((_NSLOT, chunk, d_model), jnp.float32),
            pltpu.SemaphoreType.DMA((_NSLOT,)),
            pltpu.SemaphoreType.DMA((_NSLOT, batch)),
        ],
    )
    def sc_broadcast(table_hbm, out_hbm, bufs, rsems, wsems):
        wid = lax.axis_index("s") * _NC + lax.axis_index("c")
        base = wid * rows_per_w
        nslot = min(_NSLOT, nchunk)

        def read(i, slot):
            return pltpu.make_async_copy(
                table_hbm.at[pl.ds(base + offs[i], sizes[i])],
                bufs.at[slot, pl.ds(0, sizes[i])],
                rsems.at[slot],
            )

        def write(i, slot, b):
            return pltpu.make_async_copy(
                bufs.at[slot, pl.ds(0, sizes[i])],
                out_hbm.at[b, pl.ds(base + offs[i], sizes[i])],
                wsems.at[slot, b],
            )

        ahead = max(1, nslot - 1)
        for j in range(min(ahead, nchunk)):
            read(j, j % nslot).start()
        for i in range(nchunk):
            slot = i % nslot
            read(i, slot).wait()
            nxt = i + ahead
            if nxt < nchunk:
                old = nxt - nslot
                if old >= 0:
                    for b in range(batch):
                        write(old, old % nslot, b).wait()
                read(nxt, nxt % nslot).start()
            for b in range(batch):
                write(i, slot, (b + i) % batch).start()
        for i in range(max(0, nchunk - nslot), nchunk):
            for b in range(batch):
                write(i, i % nslot, b).wait()

    return sc_broadcast


def kernel(inputs, pos_embedding):
    batch, seq_len = inputs.shape
    d_model = pos_embedding.shape[1]
    return _make_sc_broadcast(batch, seq_len, d_model)(pos_embedding)


# speedup vs baseline: 1.0025x; 1.0025x over previous
import functools

import jax
import jax.numpy as jnp
from jax import lax
from jax.experimental import pallas as pl
from jax.experimental.pallas import tpu as pltpu
from jax.experimental.pallas import tpu_sc as plsc

_INFO = plsc.get_sparse_core_info()
_NC, _NS = _INFO.num_cores, _INFO.num_subcores
_NW = _NC * _NS

_NSLOT = 2
_MAX_CHUNK = 48


def _chunk_sizes(rows):
    sizes = [_MAX_CHUNK] * (rows // _MAX_CHUNK)
    if rows % _MAX_CHUNK:
        sizes.append(rows % _MAX_CHUNK)
    return sizes


@functools.lru_cache(maxsize=None)
def _make_sc_broadcast(batch, seq_len, d_model):
    rows_per_w = seq_len // _NW
    assert rows_per_w * _NW == seq_len
    sizes = _chunk_sizes(rows_per_w)
    offs = [sum(sizes[:k]) for k in range(len(sizes))]
    nchunk = len(sizes)
    chunk = max(sizes)

    mesh = plsc.VectorSubcoreMesh(core_axis_name="c", subcore_axis_name="s")

    @functools.partial(
        pl.kernel,
        mesh=mesh,
        out_type=jax.ShapeDtypeStruct((batch, seq_len, d_model), jnp.float32),
        scratch_types=[
            pltpu.VMEM((_NSLOT, chunk, d_model), jnp.float32),
            pltpu.SemaphoreType.DMA((_NSLOT,)),
            pltpu.SemaphoreType.DMA((_NSLOT, batch)),
        ],
    )
    def sc_broadcast(table_hbm, out_hbm, bufs, rsems, wsems):
        wid = lax.axis_index("s") * _NC + lax.axis_index("c")
        base = wid * rows_per_w
        nslot = min(_NSLOT, nchunk)

        def read(i, slot):
            return pltpu.make_async_copy(
                table_hbm.at[pl.ds(base + offs[i], sizes[i])],
                bufs.at[slot, pl.ds(0, sizes[i])],
                rsems.at[slot],
            )

        def write(i, slot, b):
            return pltpu.make_async_copy(
                bufs.at[slot, pl.ds(0, sizes[i])],
                out_hbm.at[b, pl.ds(base + offs[i], sizes[i])],
                wsems.at[slot, b],
            )

        ahead = max(1, nslot - 1)
        for j in range(min(ahead, nchunk)):
            read(j, j % nslot).start()
        for i in range(nchunk):
            slot = i % nslot
            read(i, slot).wait()
            nxt = i + ahead
            if nxt < nchunk:
                old = nxt - nslot
                if old >= 0:
                    for b in range(batch):
                        write(old, old % nslot, b).wait()
                read(nxt, nxt % nslot).start()
            for b in range(batch):
                write(i, slot, (b + i) % batch).start()
        for i in range(max(0, nchunk - nslot), nchunk):
            for b in range(batch):
                write(i, i % nslot, b).wait()

    return sc_broadcast


def kernel(inputs, pos_embedding):
    batch, seq_len = inputs.shape
    d_model = pos_embedding.shape[1]
    return _make_sc_broadcast(batch, seq_len, d_model)(pos_embedding)
